# Initial kernel scaffold; baseline (speedup 1.0000x reference)
import jax
import jax.numpy as jnp
from jax.experimental import pallas as pl


def kernel(x, edge_index, W1, b1, W2, b2, W3, b3):
    raise NotImplementedError("write your pallas kernel here")


# speedup vs baseline: 69.1031x; 69.1031x over previous
import functools

import jax
import jax.numpy as jnp
from jax import lax
from jax.experimental import pallas as pl
from jax.experimental.pallas import tpu as pltpu
from jax.experimental.pallas import tpu_sc as plsc

N = 10000
D = 128
NP = 10240
NW = 32
EP = 323584
EPW = EP // NW
L = 16

_HI = jax.lax.Precision.HIGHEST


def _sc_mesh():
    return plsc.VectorSubcoreMesh(core_axis_name="c", subcore_axis_name="s")


@functools.partial(
    pl.kernel,
    out_type=jax.ShapeDtypeStruct((NW, NP), jnp.float32),
    mesh=_sc_mesh(),
    compiler_params=pltpu.CompilerParams(needs_layout_passes=False),
    scratch_types=[
        pltpu.VMEM((EPW,), jnp.int32),
        pltpu.VMEM((NP,), jnp.float32),
    ],
)
def _deg_kernel(dst_hbm, degp_hbm, dst_v, deg_v):
    wid = lax.axis_index("s") * 2 + lax.axis_index("c")
    pltpu.sync_copy(dst_hbm.at[wid], dst_v)

    zeros = jnp.zeros((L,), jnp.float32)

    def zbody(i, carry):
        deg_v[pl.ds(i * L, L)] = zeros
        return carry

    lax.fori_loop(0, NP // L, zbody, 0, unroll=False)

    ones = jnp.ones((L,), jnp.float32)

    def ebody(i, carry):
        idx = dst_v[pl.ds(i * L, L)]
        plsc.addupdate_scatter(deg_v, [idx], ones)
        return carry

    lax.fori_loop(0, EPW // L, ebody, 0, unroll=False)
    pltpu.sync_copy(deg_v, degp_hbm.at[wid])


def _make_agg(F):
    @functools.partial(
        pl.kernel,
        out_type=jax.ShapeDtypeStruct((NW, F * NP), jnp.float32),
        mesh=_sc_mesh(),
        compiler_params=pltpu.CompilerParams(needs_layout_passes=False),
        scratch_types=[
            pltpu.VMEM((EPW,), jnp.int32),
            pltpu.VMEM((EPW,), jnp.int32),
            pltpu.VMEM((F * NP,), jnp.float32),
            pltpu.VMEM((F * NP,), jnp.float32),
        ],
    )
    def _agg(g_hbm, src_hbm, dst_hbm, out_hbm, src_v, dst_v, g_v, acc_v):
        wid = lax.axis_index("s") * 2 + lax.axis_index("c")
        pltpu.sync_copy(src_hbm.at[wid], src_v)
        pltpu.sync_copy(dst_hbm.at[wid], dst_v)
        pltpu.sync_copy(g_hbm, g_v)

        zeros = jnp.zeros((L,), jnp.float32)

        def zbody(i, carry):
            acc_v[pl.ds(i * L, L)] = zeros
            return carry

        lax.fori_loop(0, F * NP // L, zbody, 0, unroll=False)

        def ebody(i, carry):
            s = src_v[pl.ds(i * L, L)]
            d = dst_v[pl.ds(i * L, L)]
            for j in range(F):
                v = plsc.load_gather(g_v, [s + (j * NP)])
                plsc.addupdate_scatter(acc_v, [d + (j * NP)], v)
            return carry

        lax.fori_loop(0, EPW // L, ebody, 0, unroll=False)
        pltpu.sync_copy(acc_v, out_hbm.at[wid])

    return _agg


_agg4 = _make_agg(4)
_agg2 = _make_agg(2)
_agg1 = _make_agg(1)


def _prep1(xp, degp, W1):
    def body(x_ref, degp_ref, w_ref, g_ref, dinv_ref):
        deg = jnp.sum(degp_ref[...], axis=0, keepdims=True) + 1.0
        dinv = 1.0 / jnp.sqrt(deg)
        h = lax.dot_general(w_ref[...], x_ref[...],
                            (((0,), (1,)), ((), ())), precision=_HI)
        g_ref[...] = h * dinv
        dinv_ref[...] = dinv

    return pl.pallas_call(
        body,
        out_shape=[
            jax.ShapeDtypeStruct((W1.shape[1], NP), jnp.float32),
            jax.ShapeDtypeStruct((1, NP), jnp.float32),
        ],
    )(xp, degp, W1)


def _prep_mid(p, g, dinv, b, W):
    def body(p_ref, g_ref, dinv_ref, b_ref, w_ref, out_ref):
        s = jnp.sum(p_ref[...], axis=0) + g_ref[...]
        o = jnp.maximum(dinv_ref[...] * s + b_ref[...], 0.0)
        h = lax.dot_general(w_ref[...], o,
                            (((0,), (0,)), ((), ())), precision=_HI)
        out_ref[...] = h * dinv_ref[...]

    return pl.pallas_call(
        body,
        out_shape=jax.ShapeDtypeStruct((W.shape[1], NP), jnp.float32),
    )(p, g, dinv, b, W)


def _final(p, g, dinv, b):
    def body(p_ref, g_ref, dinv_ref, b_ref, out_ref):
        s = jnp.sum(p_ref[...], axis=0) + g_ref[...]
        out_ref[...] = jax.nn.sigmoid(dinv_ref[...] * s + b_ref[...])

    return pl.pallas_call(
        body,
        out_shape=jax.ShapeDtypeStruct((1, NP), jnp.float32),
    )(p, g, dinv, b)


def kernel(x, edge_index, W1, b1, W2, b2, W3, b3):
    xp = jnp.concatenate([x, jnp.zeros((NP - N, D), jnp.float32)], axis=0)
    pad = jnp.full((EP - 320000,), NP - 1, jnp.int32)
    srcp = jnp.concatenate([edge_index[0], pad]).reshape(NW, EPW)
    dstp = jnp.concatenate([edge_index[1], pad]).reshape(NW, EPW)
    b1c = jnp.reshape(b1, (-1, 1))
    b2c = jnp.reshape(b2, (-1, 1))
    b3c = jnp.reshape(b3, (-1, 1))

    degp = _deg_kernel(dstp)
    g1, dinv = _prep1(xp, degp, W1)
    p1 = _agg4(g1.reshape(-1), srcp, dstp).reshape(NW, 4, NP)
    g2 = _prep_mid(p1, g1, dinv, b1c, W2)
    p2 = _agg2(g2.reshape(-1), srcp, dstp).reshape(NW, 2, NP)
    g3 = _prep_mid(p2, g2, dinv, b2c, W3)
    p3 = _agg1(g3.reshape(-1), srcp, dstp).reshape(NW, 1, NP)
    out = _final(p3, g3, dinv, b3c)
    return out[0, :N][:, None]


# speedup vs baseline: 110.1233x; 1.5936x over previous
import functools

import jax
import jax.numpy as jnp
from jax import lax
from jax.experimental import pallas as pl
from jax.experimental.pallas import tpu as pltpu
from jax.experimental.pallas import tpu_sc as plsc

N = 10000
D = 128
E = 320000
NP = 10240
NW = 32
EC = E // NW
L = 16

_HI = jax.lax.Precision.HIGHEST
_SC_PARAMS = pltpu.CompilerParams(
    needs_layout_passes=False, disable_bounds_checks=True)


def _sc_mesh():
    return plsc.VectorSubcoreMesh(core_axis_name="c", subcore_axis_name="s")


@functools.partial(
    pl.kernel,
    out_type=jax.ShapeDtypeStruct((NW, NP), jnp.float32),
    mesh=_sc_mesh(),
    compiler_params=_SC_PARAMS,
    scratch_types=[
        pltpu.VMEM((EC,), jnp.int32),
        pltpu.VMEM((NP,), jnp.float32),
        pltpu.SemaphoreType.DMA,
    ],
)
def _deg_kernel(ei_hbm, z_hbm, degp_hbm, dst_v, deg_v, sem):
    wid = lax.axis_index("s") * 2 + lax.axis_index("c")
    c1 = pltpu.async_copy(ei_hbm.at[pl.ds(E + wid * EC, EC)], dst_v, sem)
    c2 = pltpu.async_copy(z_hbm, deg_v, sem)
    c1.wait()
    c2.wait()

    ones = jnp.ones((L,), jnp.float32)

    def ebody(i, carry):
        idx = dst_v[pl.ds(i * L, L)]
        plsc.addupdate_scatter(deg_v, [idx], ones)
        return carry

    lax.fori_loop(0, EC // L, ebody, 0, unroll=5)
    pltpu.sync_copy(deg_v, degp_hbm.at[wid])


def _make_agg(F):
    @functools.partial(
        pl.kernel,
        out_type=jax.ShapeDtypeStruct((NW, F, NP), jnp.float32),
        mesh=_sc_mesh(),
        compiler_params=_SC_PARAMS,
        scratch_types=[
            pltpu.VMEM((EC,), jnp.int32),
            pltpu.VMEM((EC,), jnp.int32),
            pltpu.VMEM((F, NP), jnp.float32),
            pltpu.VMEM((F, NP), jnp.float32),
            pltpu.SemaphoreType.DMA,
        ],
    )
    def _agg(g_hbm, ei_hbm, z_hbm, out_hbm, src_v, dst_v, g_v, acc_v, sem):
        wid = lax.axis_index("s") * 2 + lax.axis_index("c")
        c1 = pltpu.async_copy(ei_hbm.at[pl.ds(wid * EC, EC)], src_v, sem)
        c2 = pltpu.async_copy(ei_hbm.at[pl.ds(E + wid * EC, EC)], dst_v, sem)
        c3 = pltpu.async_copy(g_hbm, g_v, sem)
        c4 = pltpu.async_copy(z_hbm, acc_v, sem)
        c1.wait()
        c2.wait()
        c3.wait()
        c4.wait()

        def ebody(i, carry):
            s = src_v[pl.ds(i * L, L)]
            d = dst_v[pl.ds(i * L, L)]
            for j in range(F):
                jv = jnp.full((L,), j, jnp.int32)
                v = plsc.load_gather(g_v, [jv, s])
                plsc.addupdate_scatter(acc_v, [jv, d], v)
            return carry

        lax.fori_loop(0, EC // L, ebody, 0, unroll=5)
        pltpu.sync_copy(acc_v, out_hbm.at[wid])

    return _agg


_agg4 = _make_agg(4)
_agg2 = _make_agg(2)
_agg1 = _make_agg(1)


def _matmul1(x, W1):
    def body(x_ref, w_ref, h_ref):
        h = lax.dot_general(w_ref[...], x_ref[...],
                            (((0,), (1,)), ((), ())), precision=_HI)
        h_ref[...] = jnp.pad(h, ((0, 0), (0, NP - N)))

    return pl.pallas_call(
        body,
        out_shape=jax.ShapeDtypeStruct((W1.shape[1], NP), jnp.float32),
    )(x, W1)


def _prep1(degp, h1):
    def body(degp_ref, h_ref, g_ref, dinv_ref):
        deg = jnp.sum(degp_ref[...], axis=0, keepdims=True) + 1.0
        dinv = 1.0 / jnp.sqrt(deg)
        g_ref[...] = h_ref[...] * dinv
        dinv_ref[...] = dinv

    return pl.pallas_call(
        body,
        out_shape=[
            jax.ShapeDtypeStruct(h1.shape, jnp.float32),
            jax.ShapeDtypeStruct((1, NP), jnp.float32),
        ],
    )(degp, h1)


def _prep_mid(p, g, dinv, b, W):
    def body(p_ref, g_ref, dinv_ref, b_ref, w_ref, out_ref):
        s = jnp.sum(p_ref[...], axis=0) + g_ref[...]
        o = jnp.maximum(dinv_ref[...] * s + b_ref[...], 0.0)
        h = lax.dot_general(w_ref[...], o,
                            (((0,), (0,)), ((), ())), precision=_HI)
        out_ref[...] = h * dinv_ref[...]

    return pl.pallas_call(
        body,
        out_shape=jax.ShapeDtypeStruct((W.shape[1], NP), jnp.float32),
    )(p, g, dinv, b, W)


def _final(p, g, dinv, b):
    def body(p_ref, g_ref, dinv_ref, b_ref, out_ref):
        s = jnp.sum(p_ref[...], axis=0) + g_ref[...]
        out_ref[...] = jax.nn.sigmoid(dinv_ref[...] * s + b_ref[...])

    return pl.pallas_call(
        body,
        out_shape=jax.ShapeDtypeStruct((1, NP), jnp.float32),
    )(p, g, dinv, b)


def kernel(x, edge_index, W1, b1, W2, b2, W3, b3):
    eif = edge_index.reshape(-1)
    zd = jnp.zeros((NP,), jnp.float32)
    z4 = jnp.zeros((4, NP), jnp.float32)
    z2 = jnp.zeros((2, NP), jnp.float32)
    z1 = jnp.zeros((1, NP), jnp.float32)
    b1c = jnp.reshape(b1, (-1, 1))
    b2c = jnp.reshape(b2, (-1, 1))
    b3c = jnp.reshape(b3, (-1, 1))

    degp = _deg_kernel(eif, zd)
    h1 = _matmul1(x, W1)
    g1, dinv = _prep1(degp, h1)
    p1 = _agg4(g1, eif, z4)
    g2 = _prep_mid(p1, g1, dinv, b1c, W2)
    p2 = _agg2(g2, eif, z2)
    g3 = _prep_mid(p2, g2, dinv, b2c, W3)
    p3 = _agg1(g3, eif, z1)
    out = _final(p3, g3, dinv, b3c)
    return out[0, :N][:, None]


# speedup vs baseline: 138.6999x; 1.2595x over previous
import functools

import jax
import jax.numpy as jnp
from jax import lax
from jax.experimental import pallas as pl
from jax.experimental.pallas import tpu as pltpu
from jax.experimental.pallas import tpu_sc as plsc

N = 10000
D = 128
E = 320000
NP = 10240
NW = 32
EC = E // NW
L = 16

_HI = jax.lax.Precision.HIGHEST
_SC_PARAMS = pltpu.CompilerParams(
    needs_layout_passes=False, disable_bounds_checks=True)


def _sc_mesh():
    return plsc.VectorSubcoreMesh(core_axis_name="c", subcore_axis_name="s")


@functools.partial(
    pl.kernel,
    out_type=jax.ShapeDtypeStruct((NW, NP), jnp.float32),
    mesh=_sc_mesh(),
    compiler_params=_SC_PARAMS,
    scratch_types=[
        pltpu.VMEM((EC,), jnp.int32),
        pltpu.VMEM((NP,), jnp.float32),
        pltpu.SemaphoreType.DMA,
    ],
)
def _deg_kernel(dst_hbm, degp_hbm, dst_v, deg_v, sem):
    wid = lax.axis_index("s") * 2 + lax.axis_index("c")
    c1 = pltpu.async_copy(dst_hbm.at[pl.ds(wid * EC, EC)], dst_v, sem)

    zeros = jnp.zeros((L,), jnp.float32)

    @plsc.parallel_loop(0, NP // L, unroll=8)
    def _zero(i):
        deg_v[pl.ds(i * L, L)] = zeros

    c1.wait()

    ones = jnp.ones((L,), jnp.float32)

    @plsc.parallel_loop(0, EC // L, unroll=5)
    def _edge(i):
        idx = dst_v[pl.ds(i * L, L)]
        plsc.addupdate_scatter(deg_v, [idx], ones)

    pltpu.sync_copy(deg_v, degp_hbm.at[wid])


def _make_agg(F):
    @functools.partial(
        pl.kernel,
        out_type=jax.ShapeDtypeStruct((NW, F, NP), jnp.float32),
        mesh=_sc_mesh(),
        compiler_params=_SC_PARAMS,
        scratch_types=[
            pltpu.VMEM((EC,), jnp.int32),
            pltpu.VMEM((EC,), jnp.int32),
            pltpu.VMEM((F, NP), jnp.float32),
            pltpu.VMEM((F, NP), jnp.float32),
            pltpu.SemaphoreType.DMA,
        ],
    )
    def _agg(g_hbm, src_hbm, dst_hbm, out_hbm, src_v, dst_v, g_v, acc_v, sem):
        wid = lax.axis_index("s") * 2 + lax.axis_index("c")
        c1 = pltpu.async_copy(src_hbm.at[pl.ds(wid * EC, EC)], src_v, sem)
        c2 = pltpu.async_copy(dst_hbm.at[pl.ds(wid * EC, EC)], dst_v, sem)
        c3 = pltpu.async_copy(g_hbm, g_v, sem)

        zeros = jnp.zeros((L,), jnp.float32)

        @plsc.parallel_loop(0, NP // L, unroll=8)
        def _zero(i):
            for j in range(F):
                acc_v[j, pl.ds(i * L, L)] = zeros

        c1.wait()
        c2.wait()
        c3.wait()

        @plsc.parallel_loop(0, EC // L, unroll=5)
        def _edge(i):
            s = src_v[pl.ds(i * L, L)]
            d = dst_v[pl.ds(i * L, L)]
            for j in range(F):
                jv = jnp.full((L,), j, jnp.int32)
                v = plsc.load_gather(g_v, [jv, s])
                plsc.addupdate_scatter(acc_v, [jv, d], v)

        pltpu.sync_copy(acc_v, out_hbm.at[wid])

    return _agg


_agg4 = _make_agg(4)
_agg2 = _make_agg(2)
_agg1 = _make_agg(1)


def _matmul1(x, W1):
    def body(x_ref, w_ref, h_ref):
        h = lax.dot_general(w_ref[...], x_ref[...],
                            (((0,), (1,)), ((), ())), precision=_HI)
        h_ref[...] = jnp.pad(h, ((0, 0), (0, NP - N)))

    return pl.pallas_call(
        body,
        out_shape=jax.ShapeDtypeStruct((W1.shape[1], NP), jnp.float32),
    )(x, W1)


def _prep1(degp, h1):
    def body(degp_ref, h_ref, g_ref, dinv_ref):
        deg = jnp.sum(degp_ref[...], axis=0, keepdims=True) + 1.0
        dinv = 1.0 / jnp.sqrt(deg)
        g_ref[...] = h_ref[...] * dinv
        dinv_ref[...] = dinv

    return pl.pallas_call(
        body,
        out_shape=[
            jax.ShapeDtypeStruct(h1.shape, jnp.float32),
            jax.ShapeDtypeStruct((1, NP), jnp.float32),
        ],
    )(degp, h1)


def _prep_mid(p, g, dinv, b, W):
    def body(p_ref, g_ref, dinv_ref, b_ref, w_ref, out_ref):
        s = jnp.sum(p_ref[...], axis=0) + g_ref[...]
        o = jnp.maximum(dinv_ref[...] * s + b_ref[...], 0.0)
        h = lax.dot_general(w_ref[...], o,
                            (((0,), (0,)), ((), ())), precision=_HI)
        out_ref[...] = h * dinv_ref[...]

    return pl.pallas_call(
        body,
        out_shape=jax.ShapeDtypeStruct((W.shape[1], NP), jnp.float32),
    )(p, g, dinv, b, W)


def _final(p, g, dinv, b):
    def body(p_ref, g_ref, dinv_ref, b_ref, out_ref):
        s = jnp.sum(p_ref[...], axis=0) + g_ref[...]
        out_ref[...] = jax.nn.sigmoid(dinv_ref[...] * s + b_ref[...])

    return pl.pallas_call(
        body,
        out_shape=jax.ShapeDtypeStruct((1, NP), jnp.float32),
    )(p, g, dinv, b)


def kernel(x, edge_index, W1, b1, W2, b2, W3, b3):
    src = edge_index[0]
    dst = edge_index[1]
    b1c = jnp.reshape(b1, (-1, 1))
    b2c = jnp.reshape(b2, (-1, 1))
    b3c = jnp.reshape(b3, (-1, 1))

    degp = _deg_kernel(dst)
    h1 = _matmul1(x, W1)
    g1, dinv = _prep1(degp, h1)
    p1 = _agg4(g1, src, dst)
    g2 = _prep_mid(p1, g1, dinv, b1c, W2)
    p2 = _agg2(g2, src, dst)
    g3 = _prep_mid(p2, g2, dinv, b2c, W3)
    p3 = _agg1(g3, src, dst)
    out = _final(p3, g3, dinv, b3c)
    return out[0, :N][:, None]


# speedup vs baseline: 163.2699x; 1.1771x over previous
import functools

import jax
import jax.numpy as jnp
from jax import lax
from jax.experimental import pallas as pl
from jax.experimental.pallas import tpu as pltpu
from jax.experimental.pallas import tpu_sc as plsc

N = 10000
D = 128
E = 320000
NP = 10240
NW = 32
EC = E // NW
BLK = 9984
TAIL = E - NW * BLK
EB = BLK + TAIL
L = 16

_HI = jax.lax.Precision.HIGHEST
_SC_PARAMS = pltpu.CompilerParams(
    needs_layout_passes=False, disable_bounds_checks=True)


def _sc_mesh():
    return plsc.VectorSubcoreMesh(core_axis_name="c", subcore_axis_name="s")


@functools.partial(
    pl.kernel,
    out_type=jax.ShapeDtypeStruct((NW, NP), jnp.float32),
    mesh=_sc_mesh(),
    compiler_params=_SC_PARAMS,
    scratch_types=[
        pltpu.VMEM((2, EB), jnp.int32),
        pltpu.VMEM((NP,), jnp.float32),
        pltpu.SemaphoreType.DMA,
    ],
)
def _deg_kernel(ei_hbm, degp_hbm, e_v, deg_v, sem):
    wid = lax.axis_index("s") * 2 + lax.axis_index("c")
    last = wid == NW - 1
    c1 = pltpu.async_copy(
        ei_hbm.at[:, pl.ds(wid * BLK, BLK)], e_v.at[:, pl.ds(0, BLK)], sem)
    c2 = pltpu.async_copy(
        ei_hbm.at[:, pl.ds(NW * BLK, TAIL)], e_v.at[:, pl.ds(BLK, TAIL)], sem)

    zeros = jnp.zeros((L,), jnp.float32)

    @plsc.parallel_loop(0, NP // L, unroll=8)
    def _zero(i):
        deg_v[pl.ds(i * L, L)] = zeros

    c1.wait()
    c2.wait()

    ones = jnp.ones((L,), jnp.float32)

    @plsc.parallel_loop(0, BLK // L, unroll=8)
    def _edge(i):
        idx = e_v[1, pl.ds(i * L, L)]
        plsc.addupdate_scatter(deg_v, [idx], ones)

    @pl.when(last)
    def _():
        @plsc.parallel_loop(BLK // L, EB // L, unroll=8)
        def _tail(i):
            idx = e_v[1, pl.ds(i * L, L)]
            plsc.addupdate_scatter(deg_v, [idx], ones)

    pltpu.sync_copy(deg_v, degp_hbm.at[wid])


def _make_agg(F):
    @functools.partial(
        pl.kernel,
        out_type=jax.ShapeDtypeStruct((NW, F, NP), jnp.float32),
        mesh=_sc_mesh(),
        compiler_params=_SC_PARAMS,
        scratch_types=[
            pltpu.VMEM((2, EB), jnp.int32),
            pltpu.VMEM((F, NP), jnp.float32),
            pltpu.VMEM((F, NP), jnp.float32),
            pltpu.SemaphoreType.DMA,
        ],
    )
    def _agg(g_hbm, ei_hbm, out_hbm, e_v, g_v, acc_v, sem):
        wid = lax.axis_index("s") * 2 + lax.axis_index("c")
        last = wid == NW - 1
        c1 = pltpu.async_copy(
            ei_hbm.at[:, pl.ds(wid * BLK, BLK)], e_v.at[:, pl.ds(0, BLK)], sem)
        c2 = pltpu.async_copy(
            ei_hbm.at[:, pl.ds(NW * BLK, TAIL)], e_v.at[:, pl.ds(BLK, TAIL)],
            sem)
        c3 = pltpu.async_copy(g_hbm, g_v, sem)

        zeros = jnp.zeros((L,), jnp.float32)

        @plsc.parallel_loop(0, NP // L, unroll=8)
        def _zero(i):
            for j in range(F):
                acc_v[j, pl.ds(i * L, L)] = zeros

        c1.wait()
        c2.wait()
        c3.wait()

        def _body(i):
            s = e_v[0, pl.ds(i * L, L)]
            d = e_v[1, pl.ds(i * L, L)]
            for j in range(F):
                jv = jnp.full((L,), j, jnp.int32)
                v = plsc.load_gather(g_v, [jv, s])
                plsc.addupdate_scatter(acc_v, [jv, d], v)

        @plsc.parallel_loop(0, BLK // L, unroll=8)
        def _edge(i):
            _body(i)

        @pl.when(last)
        def _():
            @plsc.parallel_loop(BLK // L, EB // L, unroll=8)
            def _tail(i):
                _body(i)

        pltpu.sync_copy(acc_v, out_hbm.at[wid])

    return _agg


_agg4 = _make_agg(4)
_agg2 = _make_agg(2)
_agg1 = _make_agg(1)


def _matmul1(x, W1):
    def body(x_ref, w_ref, h_ref):
        h = lax.dot_general(w_ref[...], x_ref[...],
                            (((0,), (1,)), ((), ())), precision=_HI)
        h_ref[:, pl.ds(0, N)] = h

    return pl.pallas_call(
        body,
        out_shape=jax.ShapeDtypeStruct((W1.shape[1], NP), jnp.float32),
    )(x, W1)


def _prep1(degp, h1):
    def body(degp_ref, h_ref, g_ref, dinv_ref):
        deg = jnp.sum(degp_ref[...], axis=0, keepdims=True) + 1.0
        dinv = 1.0 / jnp.sqrt(deg)
        g_ref[...] = h_ref[...] * dinv
        dinv_ref[...] = dinv

    return pl.pallas_call(
        body,
        out_shape=[
            jax.ShapeDtypeStruct(h1.shape, jnp.float32),
            jax.ShapeDtypeStruct((1, NP), jnp.float32),
        ],
    )(degp, h1)


def _prep_mid(p, g, dinv, b, W):
    def body(p_ref, g_ref, dinv_ref, b_ref, w_ref, out_ref):
        s = jnp.sum(p_ref[...], axis=0) + g_ref[...]
        o = jnp.maximum(dinv_ref[...] * s + b_ref[...], 0.0)
        h = lax.dot_general(w_ref[...], o,
                            (((0,), (0,)), ((), ())), precision=_HI)
        out_ref[...] = h * dinv_ref[...]

    return pl.pallas_call(
        body,
        out_shape=jax.ShapeDtypeStruct((W.shape[1], NP), jnp.float32),
    )(p, g, dinv, b, W)


def _final(p, g, dinv, b):
    def body(p_ref, g_ref, dinv_ref, b_ref, out_ref):
        s = jnp.sum(p_ref[...], axis=0) + g_ref[...]
        out_ref[...] = jax.nn.sigmoid(dinv_ref[...] * s + b_ref[...])

    return pl.pallas_call(
        body,
        out_shape=jax.ShapeDtypeStruct((1, NP), jnp.float32),
    )(p, g, dinv, b)


def kernel(x, edge_index, W1, b1, W2, b2, W3, b3):
    b1c = jnp.reshape(b1, (-1, 1))
    b2c = jnp.reshape(b2, (-1, 1))
    b3c = jnp.reshape(b3, (-1, 1))

    degp = _deg_kernel(edge_index)
    h1 = _matmul1(x, W1)
    g1, dinv = _prep1(degp, h1)
    p1 = _agg4(g1, edge_index)
    g2 = _prep_mid(p1, g1, dinv, b1c, W2)
    p2 = _agg2(g2, edge_index)
    g3 = _prep_mid(p2, g2, dinv, b2c, W3)
    p3 = _agg1(g3, edge_index)
    out = _final(p3, g3, dinv, b3c)
    return out[0, :N][:, None]
